# baseline (speedup 1.0000x reference)
import jax
import jax.numpy as jnp
from jax import lax
from jax.experimental import pallas as pl
from jax.experimental.pallas import tpu as pltpu

_ROWS = 16
_COLS = 4096
_BLK = 128
_STEPS = 4
_SCOLS = _COLS // _STEPS
_BPG = _SCOLS // _BLK


def _body(x_ref, o_ref, carry_ref):
    g = pl.program_id(0)
    x = x_ref[...].astype(jnp.bfloat16)
    i = lax.broadcasted_iota(jnp.int32, (_BLK, _BLK), 0)
    j = lax.broadcasted_iota(jnp.int32, (_BLK, _BLK), 1)
    tri = (i <= j).astype(jnp.bfloat16)
    ones_col = jnp.ones((_BLK, 1), jnp.bfloat16)

    xcat = jnp.concatenate(
        [lax.slice(x, (0, b * _BLK), (_ROWS, (b + 1) * _BLK))
         for b in range(_BPG)], axis=0)
    tots = lax.dot(xcat, ones_col, preferred_element_type=jnp.float32)

    carry = jnp.where(g == 0, jnp.zeros((_ROWS, 1), jnp.float32),
                      carry_ref[...])
    incl = [lax.slice(tots, (b * _ROWS, 0), ((b + 1) * _ROWS, 1))
            for b in range(_BPG)]
    d = 1
    while d < _BPG:
        incl = [incl[b] if b < d else incl[b] + incl[b - d]
                for b in range(_BPG)]
        d *= 2
    offs = [carry if b == 0 else carry + incl[b - 1] for b in range(_BPG)]
    carry_ref[...] = carry + incl[_BPG - 1]

    cg = lax.dot(xcat, tri, preferred_element_type=jnp.float32)
    for b in range(_BPG):
        cb = lax.slice(cg, (b * _ROWS, 0), ((b + 1) * _ROWS, _BLK))
        o_ref[:, b * _BLK:(b + 1) * _BLK] = (cb + offs[b]).astype(jnp.int32)


@jax.jit
def kernel(masks):
    x8 = masks.view(jnp.int8)
    return pl.pallas_call(
        _body,
        grid=(_STEPS,),
        in_specs=[pl.BlockSpec((_ROWS, _SCOLS), lambda g: (0, g))],
        out_specs=pl.BlockSpec((_ROWS, _SCOLS), lambda g: (0, g)),
        out_shape=jax.ShapeDtypeStruct((_ROWS, _COLS), jnp.int32),
        scratch_shapes=[pltpu.VMEM((_ROWS, 1), jnp.float32)],
        compiler_params=pltpu.CompilerParams(
            allow_input_fusion=[True],
            dimension_semantics=("arbitrary",),
        ),
    )(x8)


# speedup vs baseline: 1.7524x; 1.7524x over previous
import jax
import jax.numpy as jnp
from jax import lax
from jax.experimental import pallas as pl
from jax.experimental.pallas import tpu as pltpu

_ROWS = 16
_COLS = 4096
_BLK = 128
_NBLK = _COLS // _BLK


def _body(x_ref, o_ref):
    x = x_ref[...].astype(jnp.bfloat16)
    i = lax.broadcasted_iota(jnp.int32, (_BLK, _BLK), 0)
    j = lax.broadcasted_iota(jnp.int32, (_BLK, _BLK), 1)
    tri = (i <= j).astype(jnp.bfloat16)
    xcat = jnp.concatenate(
        [lax.slice(x, (0, b * _BLK), (_ROWS, (b + 1) * _BLK))
         for b in range(_NBLK)], axis=0)
    call = lax.dot(xcat, tri, preferred_element_type=jnp.float32)
    cbs = []
    incl = []
    for b in range(_NBLK):
        cb = lax.slice(call, (b * _ROWS, 0), ((b + 1) * _ROWS, _BLK))
        cbs.append(cb)
        incl.append(lax.slice(cb, (0, _BLK - 1), (_ROWS, _BLK)))
    d = 1
    while d < _NBLK:
        incl = [incl[b] if b < d else incl[b] + incl[b - d]
                for b in range(_NBLK)]
        d *= 2
    for b in range(_NBLK):
        ob = cbs[b] if b == 0 else cbs[b] + incl[b - 1]
        o_ref[:, b * _BLK:(b + 1) * _BLK] = ob.astype(jnp.int32)


@jax.jit
def kernel(masks):
    x8 = masks.view(jnp.int8)
    return pl.pallas_call(
        _body,
        out_shape=jax.ShapeDtypeStruct((_ROWS, _COLS), jnp.int32),
        compiler_params=pltpu.CompilerParams(allow_input_fusion=[True]),
    )(x8)
